# baseline (speedup 1.0000x reference)
import functools

import jax
import jax.numpy as jnp
from jax import lax
from jax.experimental import pallas as pl
from jax.experimental.pallas import tpu as pltpu
from jax.experimental.pallas import tpu_sc as plsc

L = 16


def _uv_body(NC, PPW, NB, VDPAD, verts_hbm, face_hbm, p2f_hbm, bary_hbm,
             out_hbm, verts_v, face_v, p2f_v, bary_v, out_v, sem):
    wid = lax.axis_index("s") * NC + lax.axis_index("c")
    base = wid * PPW

    copies = [
        pltpu.async_copy(verts_hbm.at[pl.ds(0, VDPAD)], verts_v, sem),
        pltpu.async_copy(face_hbm, face_v, sem),
        pltpu.async_copy(p2f_hbm.at[pl.ds(base, PPW)], p2f_v, sem),
        pltpu.async_copy(bary_hbm.at[pl.ds(3 * base, 3 * PPW)], bary_v, sem),
    ]
    for c in copies:
        c.wait()

    iota = lax.iota(jnp.int32, L)

    def chunk(i, carry):
        off = i * L
        f3 = p2f_v[pl.ds(off, L)] * 3
        p3 = (off + iota) * 3
        acc = [jnp.zeros((L,), jnp.float32) for _ in range(3)]
        for k in range(3):
            vk3 = plsc.load_gather(face_v, [f3 + k]) * 3
            bk = plsc.load_gather(bary_v, [p3 + k])
            for d in range(3):
                acc[d] = acc[d] + bk * plsc.load_gather(verts_v, [vk3 + d])
        for d in range(3):
            plsc.store_scatter(out_v, [p3 + d], acc[d])
        return carry

    lax.fori_loop(0, PPW // L, chunk, 0)

    outs = [pltpu.async_copy(out_v, out_hbm.at[b, pl.ds(base * 3, PPW * 3)], sem)
            for b in range(NB)]
    for c in outs:
        c.wait()


def kernel(verts_attr, pix_to_face, bary_coords, face_tensor):
    n, v, d = verts_attr.shape
    h, w = pix_to_face.shape
    P = h * w
    f = face_tensor.shape[0]

    info = plsc.get_sparse_core_info()
    NC, NS = info.num_cores, info.num_subcores
    NW = NC * NS
    PPW = P // NW

    vd = v * d
    vd_pad = min((vd + 7) // 8 * 8, n * vd)
    verts_flat = verts_attr.reshape(-1).astype(jnp.float32)
    face_flat = face_tensor.astype(jnp.int32).reshape(-1)
    p2f_flat = pix_to_face.astype(jnp.int32).reshape(-1)
    bary_flat = bary_coords.astype(jnp.float32).reshape(-1)

    mesh = plsc.VectorSubcoreMesh(core_axis_name="c", subcore_axis_name="s")
    body = functools.partial(_uv_body, NC, PPW, n, vd_pad)
    out = pl.kernel(
        body,
        out_type=jax.ShapeDtypeStruct((n, P * 3), jnp.float32),
        mesh=mesh,
        scratch_types=[
            pltpu.VMEM((vd_pad,), jnp.float32),
            pltpu.VMEM((3 * f,), jnp.int32),
            pltpu.VMEM((PPW,), jnp.int32),
            pltpu.VMEM((3 * PPW,), jnp.float32),
            pltpu.VMEM((3 * PPW,), jnp.float32),
            pltpu.SemaphoreType.DMA,
        ],
        compiler_params=pltpu.CompilerParams(needs_layout_passes=False),
    )(verts_flat, face_flat, p2f_flat, bary_flat)
    return out.reshape(n, h, w, d)


# speedup vs baseline: 24.3906x; 24.3906x over previous
import functools

import jax
import jax.numpy as jnp
from jax import lax
from jax.experimental import pallas as pl
from jax.experimental.pallas import tpu as pltpu
from jax.experimental.pallas import tpu_sc as plsc

L = 16


def _uv_body(NC, ROWS, W, NB, VP, FP, verts_hbm, face_hbm, p2f_hbm, bary_hbm,
             out_hbm, verts_v, face_v, p2f_v, bary_v, out_v, sem):
    wid = lax.axis_index("s") * NC + lax.axis_index("c")
    r0 = wid * ROWS

    copies = [
        pltpu.async_copy(verts_hbm, verts_v, sem),
        pltpu.async_copy(face_hbm, face_v, sem),
        pltpu.async_copy(p2f_hbm.at[pl.ds(r0, ROWS), :], p2f_v, sem),
    ]
    for k in range(3):
        copies.append(pltpu.async_copy(
            bary_hbm.at[k, pl.ds(r0, ROWS), :],
            bary_v.at[pl.ds(k * ROWS, ROWS), :], sem))
    for c in copies:
        c.wait()

    cpr = W // L

    def chunk(i, carry):
        row = i // cpr
        c0 = (i % cpr) * L
        f = p2f_v[row, pl.ds(c0, L)]
        acc = [jnp.zeros((L,), jnp.float32) for _ in range(3)]
        for k in range(3):
            vk = plsc.load_gather(face_v, [f + (k * FP)])
            bk = bary_v[k * ROWS + row, pl.ds(c0, L)]
            for d in range(3):
                acc[d] = acc[d] + bk * plsc.load_gather(verts_v, [vk + (d * VP)])
        for d in range(3):
            out_v[d * ROWS + row, pl.ds(c0, L)] = acc[d]
        return carry

    lax.fori_loop(0, ROWS * cpr, chunk, 0)

    outs = []
    for b in range(NB):
        for d in range(3):
            outs.append(pltpu.async_copy(
                out_v.at[pl.ds(d * ROWS, ROWS), :],
                out_hbm.at[b, d, pl.ds(r0, ROWS), :], sem))
    for c in outs:
        c.wait()


def kernel(verts_attr, pix_to_face, bary_coords, face_tensor):
    n, v, dd = verts_attr.shape
    h, w = pix_to_face.shape
    f = face_tensor.shape[0]

    info = plsc.get_sparse_core_info()
    NC, NS = info.num_cores, info.num_subcores
    NW = NC * NS
    ROWS = h // NW

    vp = (v + 127) // 128 * 128
    fp = (f + 127) // 128 * 128
    verts_pl = jnp.pad(verts_attr[0].astype(jnp.float32).T,
                       ((0, 0), (0, vp - v))).reshape(-1)
    face_pl = jnp.pad(face_tensor.astype(jnp.int32).T,
                      ((0, 0), (0, fp - f))).reshape(-1)
    p2f = pix_to_face.astype(jnp.int32)
    bary_pl = jnp.transpose(bary_coords.astype(jnp.float32), (2, 0, 1))

    mesh = plsc.VectorSubcoreMesh(core_axis_name="c", subcore_axis_name="s")
    body = functools.partial(_uv_body, NC, ROWS, w, n, vp, fp)
    out = pl.kernel(
        body,
        out_type=jax.ShapeDtypeStruct((n, 3, h, w), jnp.float32),
        mesh=mesh,
        scratch_types=[
            pltpu.VMEM((3 * vp,), jnp.float32),
            pltpu.VMEM((3 * fp,), jnp.int32),
            pltpu.VMEM((ROWS, w), jnp.int32),
            pltpu.VMEM((3 * ROWS, w), jnp.float32),
            pltpu.VMEM((3 * ROWS, w), jnp.float32),
            pltpu.SemaphoreType.DMA,
        ],
        compiler_params=pltpu.CompilerParams(needs_layout_passes=False),
    )(verts_pl, face_pl, p2f, bary_pl)
    return jnp.transpose(out, (0, 2, 3, 1))


# speedup vs baseline: 28.4454x; 1.1662x over previous
import functools

import jax
import jax.numpy as jnp
from jax import lax
from jax.experimental import pallas as pl
from jax.experimental.pallas import tpu as pltpu
from jax.experimental.pallas import tpu_sc as plsc

L = 16


def _uv_body(NC, ROWS, W, NB, VP, FP, verts_hbm, face_hbm, p2f_hbm, bary_hbm,
             out_hbm, verts_v, face_v, p2f_v, bary_v, out_v, sem):
    wid = lax.axis_index("s") * NC + lax.axis_index("c")
    r0 = wid * ROWS

    copies = [
        pltpu.async_copy(verts_hbm, verts_v, sem),
        pltpu.async_copy(face_hbm, face_v, sem),
        pltpu.async_copy(p2f_hbm.at[pl.ds(r0, ROWS), :], p2f_v, sem),
    ]
    for k in range(3):
        copies.append(pltpu.async_copy(
            bary_hbm.at[k, pl.ds(r0, ROWS), :],
            bary_v.at[pl.ds(k * ROWS, ROWS), :], sem))
    for c in copies:
        c.wait()

    cpr = W // L

    def chunk(i, carry):
        row = i // cpr
        c0 = (i % cpr) * L
        f = p2f_v[row, pl.ds(c0, L)]
        acc = [jnp.zeros((L,), jnp.float32) for _ in range(3)]
        for k in range(3):
            vk = plsc.load_gather(face_v, [f + (k * FP)])
            bk = bary_v[k * ROWS + row, pl.ds(c0, L)]
            for d in range(3):
                acc[d] = acc[d] + bk * plsc.load_gather(verts_v, [vk + (d * VP)])
        for d in range(3):
            out_v[d * ROWS + row, pl.ds(c0, L)] = acc[d]
        return carry

    outs = []
    half = ROWS // 2
    for hb in range(2):
        plsc.parallel_loop(hb * half * cpr, (hb + 1) * half * cpr,
                           unroll=4)(lambda i, c=None: chunk(i, c))
        for b in range(NB):
            for d in range(3):
                outs.append(pltpu.async_copy(
                    out_v.at[pl.ds(d * ROWS + hb * half, half), :],
                    out_hbm.at[b, d, pl.ds(r0 + hb * half, half), :], sem))
    for c in outs:
        c.wait()


def kernel(verts_attr, pix_to_face, bary_coords, face_tensor):
    n, v, dd = verts_attr.shape
    h, w = pix_to_face.shape
    f = face_tensor.shape[0]

    info = plsc.get_sparse_core_info()
    NC, NS = info.num_cores, info.num_subcores
    NW = NC * NS
    ROWS = h // NW

    vp = (v + 127) // 128 * 128
    fp = (f + 127) // 128 * 128
    verts_pl = jnp.pad(verts_attr[0].astype(jnp.float32).T,
                       ((0, 0), (0, vp - v))).reshape(-1)
    face_pl = jnp.pad(face_tensor.astype(jnp.int32).T,
                      ((0, 0), (0, fp - f))).reshape(-1)
    p2f = pix_to_face.astype(jnp.int32)
    bary_pl = jnp.transpose(bary_coords.astype(jnp.float32), (2, 0, 1))

    mesh = plsc.VectorSubcoreMesh(core_axis_name="c", subcore_axis_name="s")
    body = functools.partial(_uv_body, NC, ROWS, w, n, vp, fp)
    out = pl.kernel(
        body,
        out_type=jax.ShapeDtypeStruct((n, 3, h, w), jnp.float32),
        mesh=mesh,
        scratch_types=[
            pltpu.VMEM((3 * vp,), jnp.float32),
            pltpu.VMEM((3 * fp,), jnp.int32),
            pltpu.VMEM((ROWS, w), jnp.int32),
            pltpu.VMEM((3 * ROWS, w), jnp.float32),
            pltpu.VMEM((3 * ROWS, w), jnp.float32),
            pltpu.SemaphoreType.DMA,
        ],
        compiler_params=pltpu.CompilerParams(needs_layout_passes=False),
    )(verts_pl, face_pl, p2f, bary_pl)
    return jnp.transpose(out, (0, 2, 3, 1))
